# baseline (speedup 1.0000x reference)
import functools

import jax
import jax.numpy as jnp
from jax import lax
from jax.experimental import pallas as pl
from jax.experimental.pallas import tpu as pltpu
from jax.experimental.pallas import tpu_sc as plsc

N = 10000
E = 320000
D = 128
TE = 8
NC = 2
NS = 16
NTILE = NC * NS
CHUNK = 128
NCH = 80
E_PAD = NTILE * NCH * CHUNK
NBUF = 1
NITER = NCH // NBUF
N_PAD = 10016
BIGROWS = 640
LAST_Z = N_PAD - 15 * BIGROWS
LAST_W = N - 15 * BIGROWS


def _make_agg(Dp):
    mesh = plsc.VectorSubcoreMesh(core_axis_name="c", subcore_axis_name="s")

    @functools.partial(
        pl.kernel,
        mesh=mesh,
        compiler_params=pltpu.CompilerParams(use_tc_tiling_on_sc=False),
        out_type=jax.ShapeDtypeStruct((NC, N, Dp), jnp.float32),
        scratch_types=[
            pltpu.VMEM((CHUNK,), jnp.int32),
            pltpu.VMEM((CHUNK,), jnp.int32),
            pltpu.VMEM((CHUNK,), jnp.int32),
            pltpu.VMEM((CHUNK,), jnp.int32),
            pltpu.VMEM((CHUNK, Dp), jnp.float32),
            pltpu.VMEM((CHUNK, Dp), jnp.float32),
            pltpu.VMEM_SHARED((N_PAD, Dp), jnp.float32),
            pltpu.SemaphoreType.DMA,
            pltpu.SemaphoreType.DMA,
        ],
    )
    def agg(h_hbm, src_hbm, dst_hbm, zeros_hbm, out_hbm,
            sa0, sa1, da0, da1, b0, b1,
            acc_sh, g0, g1):
        bufs = (b0, b1)
        sidx = (sa0, sa1)
        didx = (da0, da1)
        gsem = (g0, g1)
        cid = lax.axis_index("c")
        sid = lax.axis_index("s")
        tid = sid * NC + cid

        @pl.when(sid < 15)
        def _():
            pltpu.sync_copy(zeros_hbm, acc_sh.at[pl.ds(sid * BIGROWS, BIGROWS)])

        @pl.when(sid == 15)
        def _():
            pltpu.sync_copy(zeros_hbm.at[pl.ds(0, LAST_Z)],
                            acc_sh.at[pl.ds(15 * BIGROWS, LAST_Z)])

        plsc.subcore_barrier()

        ebase = tid * NCH * CHUNK

        def body(k, carry):
            cps = []
            for b in range(NBUF):
                eb = ebase + (k * NBUF + b) * CHUNK
                pltpu.sync_copy(src_hbm.at[pl.ds(eb, CHUNK)], sidx[b])
                pltpu.sync_copy(dst_hbm.at[pl.ds(eb, CHUNK)], didx[b])
                cps.append(
                    pltpu.async_copy(h_hbm.at[sidx[b]], bufs[b], gsem[b]))
            for b in range(NBUF):
                cps[b].wait()
                pltpu.sync_copy(bufs[b], acc_sh.at[didx[b]], add=True)
            return carry

        lax.fori_loop(0, NITER, body, 0)

        plsc.subcore_barrier()

        @pl.when(sid < 15)
        def _():
            pltpu.sync_copy(acc_sh.at[pl.ds(sid * BIGROWS, BIGROWS)],
                            out_hbm.at[cid, pl.ds(sid * BIGROWS, BIGROWS)])

        @pl.when(sid == 15)
        def _():
            pltpu.sync_copy(acc_sh.at[pl.ds(15 * BIGROWS, LAST_W)],
                            out_hbm.at[cid, pl.ds(15 * BIGROWS, LAST_W)])

    return agg


_agg_144 = _make_agg(144)
_agg_128 = _make_agg(128)


def _mlp_body(h_ref, p_ref, w1_ref, b1_ref, w2_ref, b2_ref, o_ref):
    z = h_ref[...] + p_ref[0] + p_ref[1]
    y = jnp.maximum(
        jnp.dot(z, w1_ref[...], preferred_element_type=jnp.float32) + b1_ref[...], 0.0)
    o_ref[...] = jnp.maximum(
        jnp.dot(y, w2_ref[...], preferred_element_type=jnp.float32) + b2_ref[...], 0.0)


def _mlp(h, parts, w1, b1, w2, b2):
    n, dp = h.shape
    blk = 2000
    full = lambda *shape: pl.BlockSpec(shape, lambda i: (0,) * len(shape))
    return pl.pallas_call(
        _mlp_body,
        grid=(n // blk,),
        in_specs=[
            pl.BlockSpec((blk, dp), lambda i: (i, 0)),
            pl.BlockSpec((NC, blk, dp), lambda i: (0, i, 0)),
            full(dp, D), full(1, D), full(D, D), full(1, D),
        ],
        out_specs=pl.BlockSpec((blk, D), lambda i: (i, 0)),
        out_shape=jax.ShapeDtypeStruct((n, D), jnp.float32),
    )(h, parts, w1, b1.reshape(1, D), w2, b2.reshape(1, D))


def _mlp_readout_body(h_ref, p_ref, w1_ref, b1_ref, w2_ref, b2_ref,
                      wo_ref, bo_ref, o_ref):
    z = h_ref[...] + p_ref[0] + p_ref[1]
    y = jnp.maximum(
        jnp.dot(z, w1_ref[...], preferred_element_type=jnp.float32) + b1_ref[...], 0.0)
    h3 = jnp.maximum(
        jnp.dot(y, w2_ref[...], preferred_element_type=jnp.float32) + b2_ref[...], 0.0)
    o_ref[...] = jnp.dot(h3, wo_ref[...], preferred_element_type=jnp.float32) + bo_ref[...]


def _mlp_readout(h, parts, w1, b1, w2, b2, wo, bo):
    n = h.shape[0]
    return pl.pallas_call(
        _mlp_readout_body,
        out_shape=jax.ShapeDtypeStruct((n, 1), jnp.float32),
    )(h, parts, w1, b1.reshape(1, D), w2, b2.reshape(1, D), wo, bo.reshape(1, 1))


def kernel(x_user, x_product, x_seller, x_brand, x_category, edge_index, type_emb,
           W1_0, b1_0, W2_0, b2_0, W1_1, b1_1, W2_1, b2_1, W1_2, b1_2, W2_2, b2_2,
           W_out, b_out):
    counts = [3000, 2500, 1500, 1500, 1500]
    x_all = jnp.concatenate([x_user, x_product, x_seller, x_brand, x_category], axis=0)
    te = jnp.concatenate(
        [jnp.broadcast_to(type_emb[i], (n, TE)) for i, n in enumerate(counts)], axis=0)
    h0 = jnp.concatenate(
        [x_all, te, jnp.zeros((N, 144 - D - TE), jnp.float32)], axis=1)
    w1_0p = jnp.concatenate([W1_0, jnp.zeros((144 - D - TE, D), jnp.float32)], axis=0)

    per_tile = E // NTILE
    pad_tile = NCH * CHUNK - per_tile
    dummy_dst = jnp.broadcast_to(
        N + (jnp.arange(pad_tile, dtype=jnp.int32) % 16), (NTILE, pad_tile))
    src = jnp.concatenate(
        [edge_index[0].reshape(NTILE, per_tile),
         jnp.zeros((NTILE, pad_tile), jnp.int32)], axis=1).reshape(-1)
    dst = jnp.concatenate(
        [edge_index[1].reshape(NTILE, per_tile), dummy_dst], axis=1).reshape(-1)
    z144 = jnp.zeros((BIGROWS, 144), jnp.float32)
    z128 = jnp.zeros((BIGROWS, 128), jnp.float32)

    p0 = _agg_144(h0, src, dst, z144)
    h1 = _mlp(h0, p0, w1_0p, b1_0, W2_0, b2_0)

    p1 = _agg_128(h1, src, dst, z128)
    h2 = _mlp(h1, p1, W1_1, b1_1, W2_1, b2_1)

    p2 = _agg_128(h2, src, dst, z128)
    h2s = lax.slice(h2, (3000, 0), (5500, D))
    p2s = lax.slice(p2, (0, 3000, 0), (NC, 5500, D))
    out = _mlp_readout(h2s, p2s, W1_2, b1_2, W2_2, b2_2, W_out, b_out)
    return out.reshape(2500)


# speedup vs baseline: 1.1388x; 1.1388x over previous
import functools

import jax
import jax.numpy as jnp
from jax import lax
from jax.experimental import pallas as pl
from jax.experimental.pallas import tpu as pltpu
from jax.experimental.pallas import tpu_sc as plsc

N = 10000
E = 320000
D = 128
TE = 8
NC = 2
NS = 16
NTILE = NC * NS
CHUNK = 128
NCH = 80
E_PAD = NTILE * NCH * CHUNK
NBUF = 2
NITER = NCH // NBUF
N_PAD = 10128
BIGROWS = 640
LAST_Z = N_PAD - 15 * BIGROWS
LAST_W = N - 15 * BIGROWS


def _make_agg(Dp):
    mesh = plsc.VectorSubcoreMesh(core_axis_name="c", subcore_axis_name="s")

    @functools.partial(
        pl.kernel,
        mesh=mesh,
        compiler_params=pltpu.CompilerParams(use_tc_tiling_on_sc=False),
        out_type=jax.ShapeDtypeStruct((NC, N, Dp), jnp.float32),
        scratch_types=[
            pltpu.VMEM((CHUNK,), jnp.int32),
            pltpu.VMEM((CHUNK,), jnp.int32),
            pltpu.VMEM((CHUNK,), jnp.int32),
            pltpu.VMEM((CHUNK,), jnp.int32),
            pltpu.VMEM((CHUNK, Dp), jnp.float32),
            pltpu.VMEM((CHUNK, Dp), jnp.float32),
            pltpu.VMEM_SHARED((N_PAD, Dp), jnp.float32),
            pltpu.SemaphoreType.DMA,
            pltpu.SemaphoreType.DMA,
        ],
    )
    def agg(h_hbm, src_hbm, dst_hbm, zeros_hbm, out_hbm,
            sa0, sa1, da0, da1, b0, b1,
            acc_sh, g0, g1):
        bufs = (b0, b1)
        sidx = (sa0, sa1)
        didx = (da0, da1)
        gsem = (g0, g1)
        cid = lax.axis_index("c")
        sid = lax.axis_index("s")
        tid = sid * NC + cid

        @pl.when(sid < 15)
        def _():
            pltpu.sync_copy(zeros_hbm, acc_sh.at[pl.ds(sid * BIGROWS, BIGROWS)])

        @pl.when(sid == 15)
        def _():
            pltpu.sync_copy(zeros_hbm.at[pl.ds(0, LAST_Z)],
                            acc_sh.at[pl.ds(15 * BIGROWS, LAST_Z)])

        plsc.subcore_barrier()

        ebase = tid * NCH * CHUNK

        def body(k, carry):
            cps = []
            for b in range(NBUF):
                eb = ebase + (k * NBUF + b) * CHUNK
                pltpu.sync_copy(src_hbm.at[pl.ds(eb, CHUNK)], sidx[b])
                pltpu.sync_copy(dst_hbm.at[pl.ds(eb, CHUNK)], didx[b])
                cps.append(
                    pltpu.async_copy(h_hbm.at[sidx[b]], bufs[b], gsem[b]))
            for b in range(NBUF):
                cps[b].wait()
                pltpu.sync_copy(bufs[b], acc_sh.at[didx[b]], add=True)
            return carry

        lax.fori_loop(0, NITER, body, 0)

        plsc.subcore_barrier()

        @pl.when(sid < 15)
        def _():
            pltpu.sync_copy(acc_sh.at[pl.ds(sid * BIGROWS, BIGROWS)],
                            out_hbm.at[cid, pl.ds(sid * BIGROWS, BIGROWS)])

        @pl.when(sid == 15)
        def _():
            pltpu.sync_copy(acc_sh.at[pl.ds(15 * BIGROWS, LAST_W)],
                            out_hbm.at[cid, pl.ds(15 * BIGROWS, LAST_W)])

    return agg


_agg_144 = _make_agg(144)
_agg_128 = _make_agg(128)


def _mlp_body(h_ref, p_ref, w1_ref, b1_ref, w2_ref, b2_ref, o_ref):
    z = h_ref[...] + p_ref[0] + p_ref[1]
    y = jnp.maximum(
        jnp.dot(z, w1_ref[...], preferred_element_type=jnp.float32) + b1_ref[...], 0.0)
    o_ref[...] = jnp.maximum(
        jnp.dot(y, w2_ref[...], preferred_element_type=jnp.float32) + b2_ref[...], 0.0)


def _mlp(h, parts, w1, b1, w2, b2):
    n, dp = h.shape
    blk = 2000
    full = lambda *shape: pl.BlockSpec(shape, lambda i: (0,) * len(shape))
    return pl.pallas_call(
        _mlp_body,
        grid=(n // blk,),
        in_specs=[
            pl.BlockSpec((blk, dp), lambda i: (i, 0)),
            pl.BlockSpec((NC, blk, dp), lambda i: (0, i, 0)),
            full(dp, D), full(1, D), full(D, D), full(1, D),
        ],
        out_specs=pl.BlockSpec((blk, D), lambda i: (i, 0)),
        out_shape=jax.ShapeDtypeStruct((n, D), jnp.float32),
    )(h, parts, w1, b1.reshape(1, D), w2, b2.reshape(1, D))


def _mlp_readout_body(h_ref, p_ref, w1_ref, b1_ref, w2_ref, b2_ref,
                      wo_ref, bo_ref, o_ref):
    z = h_ref[...] + p_ref[0] + p_ref[1]
    y = jnp.maximum(
        jnp.dot(z, w1_ref[...], preferred_element_type=jnp.float32) + b1_ref[...], 0.0)
    h3 = jnp.maximum(
        jnp.dot(y, w2_ref[...], preferred_element_type=jnp.float32) + b2_ref[...], 0.0)
    o_ref[...] = jnp.dot(h3, wo_ref[...], preferred_element_type=jnp.float32) + bo_ref[...]


def _mlp_readout(h, parts, w1, b1, w2, b2, wo, bo):
    n = h.shape[0]
    return pl.pallas_call(
        _mlp_readout_body,
        out_shape=jax.ShapeDtypeStruct((n, 1), jnp.float32),
    )(h, parts, w1, b1.reshape(1, D), w2, b2.reshape(1, D), wo, bo.reshape(1, 1))


def kernel(x_user, x_product, x_seller, x_brand, x_category, edge_index, type_emb,
           W1_0, b1_0, W2_0, b2_0, W1_1, b1_1, W2_1, b2_1, W1_2, b1_2, W2_2, b2_2,
           W_out, b_out):
    counts = [3000, 2500, 1500, 1500, 1500]
    x_all = jnp.concatenate([x_user, x_product, x_seller, x_brand, x_category], axis=0)
    te = jnp.concatenate(
        [jnp.broadcast_to(type_emb[i], (n, TE)) for i, n in enumerate(counts)], axis=0)
    h0 = jnp.concatenate(
        [x_all, te, jnp.zeros((N, 144 - D - TE), jnp.float32)], axis=1)
    w1_0p = jnp.concatenate([W1_0, jnp.zeros((144 - D - TE, D), jnp.float32)], axis=0)

    per_tile = E // NTILE
    pad_tile = NCH * CHUNK - per_tile
    dummy_dst = jnp.broadcast_to(
        N + (jnp.arange(pad_tile, dtype=jnp.int32) % 128), (NTILE, pad_tile))
    src = jnp.concatenate(
        [edge_index[0].reshape(NTILE, per_tile),
         jnp.zeros((NTILE, pad_tile), jnp.int32)], axis=1).reshape(-1)
    dst = jnp.concatenate(
        [edge_index[1].reshape(NTILE, per_tile), dummy_dst], axis=1).reshape(-1)
    z144 = jnp.zeros((BIGROWS, 144), jnp.float32)
    z128 = jnp.zeros((BIGROWS, 128), jnp.float32)

    p0 = _agg_144(h0, src, dst, z144)
    h1 = _mlp(h0, p0, w1_0p, b1_0, W2_0, b2_0)

    p1 = _agg_128(h1, src, dst, z128)
    h2 = _mlp(h1, p1, W1_1, b1_1, W2_1, b2_1)

    p2 = _agg_128(h2, src, dst, z128)
    h2s = lax.slice(h2, (3000, 0), (5500, D))
    p2s = lax.slice(p2, (0, 3000, 0), (NC, 5500, D))
    out = _mlp_readout(h2s, p2s, W1_2, b1_2, W2_2, b2_2, W_out, b_out)
    return out.reshape(2500)


# speedup vs baseline: 2.6268x; 2.3067x over previous
import functools

import jax
import jax.numpy as jnp
from jax import lax
from jax.experimental import pallas as pl
from jax.experimental.pallas import tpu as pltpu
from jax.experimental.pallas import tpu_sc as plsc

N = 10000
E = 320000
D = 128
TE = 8
NC = 2
NS = 16
NTILE = NC * NS
CHUNK = 128
EPT = E // NTILE
NFULL = EPT // CHUNK
TAIL = EPT - NFULL * CHUNK
NBUF = 2
NITER = NFULL // NBUF
N_PAD = 10128
BIGROWS = 640
LAST_Z = N_PAD - 15 * BIGROWS
LAST_W = N - 15 * BIGROWS


def _make_agg(Dp):
    mesh = plsc.VectorSubcoreMesh(core_axis_name="c", subcore_axis_name="s")

    @functools.partial(
        pl.kernel,
        mesh=mesh,
        compiler_params=pltpu.CompilerParams(use_tc_tiling_on_sc=False),
        out_type=jax.ShapeDtypeStruct((NC, N, Dp), jnp.float32),
        scratch_types=[
            pltpu.VMEM((CHUNK,), jnp.int32),
            pltpu.VMEM((CHUNK,), jnp.int32),
            pltpu.VMEM((CHUNK,), jnp.int32),
            pltpu.VMEM((CHUNK,), jnp.int32),
            pltpu.VMEM((CHUNK, Dp), jnp.float32),
            pltpu.VMEM((CHUNK, Dp), jnp.float32),
            pltpu.VMEM((TAIL,), jnp.int32),
            pltpu.VMEM((TAIL,), jnp.int32),
            pltpu.VMEM((TAIL, Dp), jnp.float32),
            pltpu.VMEM_SHARED((N_PAD, Dp), jnp.float32),
            pltpu.SemaphoreType.DMA,
            pltpu.SemaphoreType.DMA,
        ],
    )
    def agg(h_hbm, src_hbm, dst_hbm, zeros_hbm, out_hbm,
            sa0, sa1, da0, da1, b0, b1, sat, dat, bt,
            acc_sh, g0, g1):
        bufs = (b0, b1)
        sidx = (sa0, sa1)
        didx = (da0, da1)
        gsem = (g0, g1)
        cid = lax.axis_index("c")
        sid = lax.axis_index("s")
        tid = sid * NC + cid

        @pl.when(sid < 15)
        def _():
            pltpu.sync_copy(zeros_hbm, acc_sh.at[pl.ds(sid * BIGROWS, BIGROWS)])

        @pl.when(sid == 15)
        def _():
            pltpu.sync_copy(zeros_hbm.at[pl.ds(0, LAST_Z)],
                            acc_sh.at[pl.ds(15 * BIGROWS, LAST_Z)])

        plsc.subcore_barrier()

        ebase = tid * EPT

        def body(k, carry):
            cps = []
            for b in range(NBUF):
                eb = ebase + (k * NBUF + b) * CHUNK
                pltpu.sync_copy(src_hbm.at[pl.ds(eb, CHUNK)], sidx[b])
                pltpu.sync_copy(dst_hbm.at[pl.ds(eb, CHUNK)], didx[b])
                cps.append(
                    pltpu.async_copy(h_hbm.at[sidx[b]], bufs[b], gsem[b]))
            for b in range(NBUF):
                cps[b].wait()
                pltpu.sync_copy(bufs[b], acc_sh.at[didx[b]], add=True)
            return carry

        lax.fori_loop(0, NITER, body, 0)

        bt_off = ebase + NFULL * CHUNK
        pltpu.sync_copy(src_hbm.at[pl.ds(bt_off, TAIL)], sat)
        pltpu.sync_copy(dst_hbm.at[pl.ds(bt_off, TAIL)], dat)
        pltpu.async_copy(h_hbm.at[sat], bt, g0).wait()
        pltpu.sync_copy(bt, acc_sh.at[dat], add=True)

        plsc.subcore_barrier()

        @pl.when(sid < 15)
        def _():
            pltpu.sync_copy(acc_sh.at[pl.ds(sid * BIGROWS, BIGROWS)],
                            out_hbm.at[cid, pl.ds(sid * BIGROWS, BIGROWS)])

        @pl.when(sid == 15)
        def _():
            pltpu.sync_copy(acc_sh.at[pl.ds(15 * BIGROWS, LAST_W)],
                            out_hbm.at[cid, pl.ds(15 * BIGROWS, LAST_W)])

    return agg


_agg_144 = _make_agg(144)
_agg_128 = _make_agg(128)


def _mlp_body(h_ref, p_ref, w1_ref, b1_ref, w2_ref, b2_ref, o_ref):
    z = h_ref[...] + p_ref[0] + p_ref[1]
    y = jnp.maximum(
        jnp.dot(z, w1_ref[...], preferred_element_type=jnp.float32) + b1_ref[...], 0.0)
    o_ref[...] = jnp.maximum(
        jnp.dot(y, w2_ref[...], preferred_element_type=jnp.float32) + b2_ref[...], 0.0)


def _mlp(h, parts, w1, b1, w2, b2):
    n, dp = h.shape
    blk = 2000
    full = lambda *shape: pl.BlockSpec(shape, lambda i: (0,) * len(shape))
    return pl.pallas_call(
        _mlp_body,
        grid=(n // blk,),
        in_specs=[
            pl.BlockSpec((blk, dp), lambda i: (i, 0)),
            pl.BlockSpec((NC, blk, dp), lambda i: (0, i, 0)),
            full(dp, D), full(1, D), full(D, D), full(1, D),
        ],
        out_specs=pl.BlockSpec((blk, D), lambda i: (i, 0)),
        out_shape=jax.ShapeDtypeStruct((n, D), jnp.float32),
    )(h, parts, w1, b1.reshape(1, D), w2, b2.reshape(1, D))


def _mlp_readout_body(h_ref, p_ref, w1_ref, b1_ref, w2_ref, b2_ref,
                      wo_ref, bo_ref, o_ref):
    z = h_ref[...] + p_ref[0] + p_ref[1]
    y = jnp.maximum(
        jnp.dot(z, w1_ref[...], preferred_element_type=jnp.float32) + b1_ref[...], 0.0)
    h3 = jnp.maximum(
        jnp.dot(y, w2_ref[...], preferred_element_type=jnp.float32) + b2_ref[...], 0.0)
    o_ref[...] = jnp.dot(h3, wo_ref[...], preferred_element_type=jnp.float32) + bo_ref[...]


def _mlp_readout(h, parts, w1, b1, w2, b2, wo, bo):
    n = h.shape[0]
    return pl.pallas_call(
        _mlp_readout_body,
        out_shape=jax.ShapeDtypeStruct((n, 1), jnp.float32),
    )(h, parts, w1, b1.reshape(1, D), w2, b2.reshape(1, D), wo, bo.reshape(1, 1))


def kernel(x_user, x_product, x_seller, x_brand, x_category, edge_index, type_emb,
           W1_0, b1_0, W2_0, b2_0, W1_1, b1_1, W2_1, b2_1, W1_2, b1_2, W2_2, b2_2,
           W_out, b_out):
    counts = [3000, 2500, 1500, 1500, 1500]
    x_all = jnp.concatenate([x_user, x_product, x_seller, x_brand, x_category], axis=0)
    te = jnp.concatenate(
        [jnp.broadcast_to(type_emb[i], (n, TE)) for i, n in enumerate(counts)], axis=0)
    h0 = jnp.concatenate(
        [x_all, te, jnp.zeros((N, 144 - D - TE), jnp.float32)], axis=1)
    w1_0p = jnp.concatenate([W1_0, jnp.zeros((144 - D - TE, D), jnp.float32)], axis=0)

    src = edge_index[0]
    dst = edge_index[1]
    z144 = jnp.zeros((BIGROWS, 144), jnp.float32)
    z128 = jnp.zeros((BIGROWS, 128), jnp.float32)

    p0 = _agg_144(h0, src, dst, z144)
    h1 = _mlp(h0, p0, w1_0p, b1_0, W2_0, b2_0)

    p1 = _agg_128(h1, src, dst, z128)
    h2 = _mlp(h1, p1, W1_1, b1_1, W2_1, b2_1)

    p2 = _agg_128(h2, src, dst, z128)
    h2s = lax.slice(h2, (3000, 0), (5500, D))
    p2s = lax.slice(p2, (0, 3000, 0), (NC, 5500, D))
    out = _mlp_readout(h2s, p2s, W1_2, b1_2, W2_2, b2_2, W_out, b_out)
    return out.reshape(2500)
